# baseline (speedup 1.0000x reference)
import numpy as np
import jax
import jax.numpy as jnp
from jax.experimental import pallas as pl
from jax.experimental.pallas import tpu as pltpu

_MS = 100
_N = 13
_BF = jnp.bfloat16
_F32 = jnp.float32


def _dot_bf_t(a, b):
    return jax.lax.dot_general(a.astype(_BF), b.astype(_BF),
                               (((1,), (1,)), ((), ())),
                               preferred_element_type=_F32)


def _dot_bf(a, b):
    return jax.lax.dot_general(a.astype(_BF), b.astype(_BF),
                               (((1,), (0,)), ((), ())),
                               preferred_element_type=_F32)


def _selu(x):
    alpha = 1.6732632423543772848170429916717
    scale = 1.0507009873554804934193349852946
    neg = alpha * (jnp.exp(jnp.minimum(x, 0.0)) - 1.0)
    return scale * jnp.where(x > 0, x, neg)


def _mlp5(x, w_ref, b_ref):
    for i in range(5):
        x = _selu(_dot_bf_t(x, w_ref[i]) + b_ref[i:i + 1, :])
    return x


def _body(h0_ref, ebc_ref, mwt_ref, m_ref, maskn_ref, wih_ref, whh_ref,
          bih_ref, bhh_ref, m1w_ref, m1b_ref, m2w_ref, m2b_ref, m3w_ref,
          m3b_ref, mtw_ref, mtb_ref, w1_ref, b1_ref, w2_ref, b2_ref,
          out_ref, we_s, mt_s):
    for d in range(_N):
        p0 = ebc_ref[4 * d + 0] * mwt_ref[0]
        p1 = ebc_ref[4 * d + 1] * mwt_ref[1]
        p2 = ebc_ref[4 * d + 2] * mwt_ref[2]
        p3 = ebc_ref[4 * d + 3] * mwt_ref[3]
        we_s[d] = ((p0 + p2) + (p1 + p3)).astype(_BF)

    mbf = m_ref[...].astype(_BF)
    maskn = maskn_ref[...] > 0.0
    h = h0_ref[...]

    for _ in range(3):
        rbh = h.astype(_BF)
        rows = []
        for d in range(_N):
            rows.append(jax.lax.dot_general(
                rbh[d:d + 1, :], we_s[d], (((1,), (1,)), ((), ())),
                preferred_element_type=_F32))
        mt13 = jnp.concatenate(rows, axis=0).astype(_BF)
        for n_ in range(_N):
            mt_s[13 * n_:13 * (n_ + 1), :] = mt13[:, 100 * n_:100 * (n_ + 1)]
        msgs = jax.lax.dot_general(mbf, mt_s[...],
                                   (((1,), (0,)), ((), ())),
                                   preferred_element_type=_F32)

        gi_r = _dot_bf_t(msgs, wih_ref[0]) + bih_ref[0:1, :]
        gi_z = _dot_bf_t(msgs, wih_ref[1]) + bih_ref[1:2, :]
        gi_n = _dot_bf_t(msgs, wih_ref[2]) + bih_ref[2:3, :]
        gh_r = _dot_bf_t(h, whh_ref[0]) + bhh_ref[0:1, :]
        gh_z = _dot_bf_t(h, whh_ref[1]) + bhh_ref[1:2, :]
        gh_n = _dot_bf_t(h, whh_ref[2]) + bhh_ref[2:3, :]
        r = jax.nn.sigmoid(gi_r + gh_r)
        z = jax.nn.sigmoid(gi_z + gh_z)
        n = jnp.tanh(gi_n + r * gh_n)
        h_new = (1.0 - z) * n + z * h
        h = jnp.where(maskn, h_new, h)

    g = jnp.sum(h, axis=0, keepdims=True)
    fadd1 = _mlp5(h, m1w_ref, m1b_ref)
    fconn1 = _mlp5(h, m2w_ref, m2b_ref)
    x = jnp.concatenate([fadd1, g, fconn1, g], axis=0)
    y = _mlp5(x, m3w_ref, m3b_ref)
    fterm = _mlp5(g, mtw_ref, mtb_ref)

    rows = [y[i:i + 1, :] for i in range(28)] + [fterm]
    cat = jnp.concatenate(rows, axis=1)
    o1 = _selu(_dot_bf_t(cat, w1_ref[...]) + b1_ref[...])
    out_ref[...] = _selu(_dot_bf_t(o1, w2_ref[...]) + b2_ref[...])


_EN_EQ = np.arange(_N)[:, None] == (np.arange(_N * _N)[None, :] // _N)


def kernel(nodes, edges, message_weights, gru_w_ih, gru_w_hh, gru_b_ih,
           gru_b_hh, mlp1_W, mlp1_b, mlp2_W, mlp2_b, mlp3_W, mlp3_b,
           mlpt_W, mlpt_b, fl_W1, fl_b1, fl_W2, fl_b2):
    adjacency = jnp.sum(edges, axis=3)
    mask_e = adjacency.reshape(-1) != 0
    m_mat = jnp.where(jnp.asarray(_EN_EQ) & mask_e[None, :], 1.0, 0.0)
    maskn = jnp.broadcast_to(
        (jnp.sum(adjacency, axis=-1).reshape(-1) != 0).astype(_F32)[:, None],
        (_N, _MS))
    ebc = jnp.repeat(jnp.transpose(edges[0], (1, 2, 0)).reshape(52, _N),
                     _MS, axis=1)[:, :, None]
    mwt13 = jnp.tile(jnp.transpose(message_weights, (2, 0, 1)),
                     (1, _N, 1))
    out = pl.pallas_call(
        _body,
        out_shape=jax.ShapeDtypeStruct((1, 989), _F32),
        scratch_shapes=[pltpu.VMEM((_N, _N * _MS, _MS), _BF),
                        pltpu.VMEM((_N * _N, _MS), _BF)],
    )(nodes[0], ebc, mwt13, m_mat, maskn,
      gru_w_ih.reshape(3, _MS, _MS), gru_w_hh.reshape(3, _MS, _MS),
      gru_b_ih.reshape(3, _MS), gru_b_hh.reshape(3, _MS),
      mlp1_W, mlp1_b, mlp2_W, mlp2_b, mlp3_W, mlp3_b, mlpt_W, mlpt_b,
      fl_W1, fl_b1.reshape(1, 500), fl_W2, fl_b2.reshape(1, 989))
    return out.reshape(989)


# speedup vs baseline: 2.1448x; 2.1448x over previous
import numpy as np
import jax
import jax.numpy as jnp
from jax.experimental import pallas as pl
from jax.experimental.pallas import tpu as pltpu

_MS = 100
_N = 13
_BF = jnp.bfloat16
_F32 = jnp.float32


def _dot_bf_t(a, b):
    return jax.lax.dot_general(a.astype(_BF), b.astype(_BF),
                               (((1,), (1,)), ((), ())),
                               preferred_element_type=_F32)


def _dot_bf(a, b):
    return jax.lax.dot_general(a.astype(_BF), b.astype(_BF),
                               (((1,), (0,)), ((), ())),
                               preferred_element_type=_F32)


def _selu(x):
    alpha = 1.6732632423543772848170429916717
    scale = 1.0507009873554804934193349852946
    neg = alpha * (jnp.exp(jnp.minimum(x, 0.0)) - 1.0)
    return scale * jnp.where(x > 0, x, neg)


def _mlp5(x, w_ref, b_ref):
    for i in range(5):
        x = _selu(_dot_bf_t(x, w_ref[i]) + b_ref[i:i + 1, :])
    return x


def _body(h0_ref, esc_ref, mw_ref, m_ref, maskn_ref, wih_ref, whh_ref,
          bih_ref, bhh_ref, m1w_ref, m1b_ref, m2w_ref, m2b_ref, m3w_ref,
          m3b_ref, mtw_ref, mtb_ref, w1_ref, b1_ref, w2_ref, b2_ref,
          out_ref, we_s, mt_s):
    mw0 = mw_ref[0]
    mw1 = mw_ref[1]
    mw2 = mw_ref[2]
    mw3 = mw_ref[3]
    for d in range(_N):
        for n_ in range(_N):
            p0 = esc_ref[n_, d, 0] * mw0
            p1 = esc_ref[n_, d, 1] * mw1
            p2 = esc_ref[n_, d, 2] * mw2
            p3 = esc_ref[n_, d, 3] * mw3
            we_s[d, _MS * n_:_MS * (n_ + 1), :] = (
                (p0 + p2) + (p1 + p3)).astype(_BF)

    mbf = m_ref[...].astype(_BF)
    maskn = maskn_ref[...] > 0.0
    h = h0_ref[...]

    for _ in range(3):
        rbh = h.astype(_BF)
        rows = []
        for d in range(_N):
            rows.append(jax.lax.dot_general(
                rbh[d:d + 1, :], we_s[d], (((1,), (1,)), ((), ())),
                preferred_element_type=_F32))
        mt13 = jnp.concatenate(rows, axis=0).astype(_BF)
        for n_ in range(_N):
            mt_s[13 * n_:13 * (n_ + 1), :] = mt13[:, 100 * n_:100 * (n_ + 1)]
        msgs = jax.lax.dot_general(mbf, mt_s[...],
                                   (((1,), (0,)), ((), ())),
                                   preferred_element_type=_F32)

        gi_r = _dot_bf_t(msgs, wih_ref[0]) + bih_ref[0:1, :]
        gi_z = _dot_bf_t(msgs, wih_ref[1]) + bih_ref[1:2, :]
        gi_n = _dot_bf_t(msgs, wih_ref[2]) + bih_ref[2:3, :]
        gh_r = _dot_bf_t(h, whh_ref[0]) + bhh_ref[0:1, :]
        gh_z = _dot_bf_t(h, whh_ref[1]) + bhh_ref[1:2, :]
        gh_n = _dot_bf_t(h, whh_ref[2]) + bhh_ref[2:3, :]
        r = jax.nn.sigmoid(gi_r + gh_r)
        z = jax.nn.sigmoid(gi_z + gh_z)
        n = jnp.tanh(gi_n + r * gh_n)
        h_new = (1.0 - z) * n + z * h
        h = jnp.where(maskn, h_new, h)

    g = jnp.sum(h, axis=0, keepdims=True)
    fadd1 = _mlp5(h, m1w_ref, m1b_ref)
    fconn1 = _mlp5(h, m2w_ref, m2b_ref)
    x = jnp.concatenate([fadd1, g, fconn1, g], axis=0)
    y = _mlp5(x, m3w_ref, m3b_ref)
    fterm = _mlp5(g, mtw_ref, mtb_ref)

    rows = [y[i:i + 1, :] for i in range(28)] + [fterm]
    cat = jnp.concatenate(rows, axis=1)
    o1 = _selu(_dot_bf_t(cat, w1_ref[...]) + b1_ref[...])
    out_ref[...] = _selu(_dot_bf_t(o1, w2_ref[...]) + b2_ref[...])


_EN_EQ = np.arange(_N)[:, None] == (np.arange(_N * _N)[None, :] // _N)


def kernel(nodes, edges, message_weights, gru_w_ih, gru_w_hh, gru_b_ih,
           gru_b_hh, mlp1_W, mlp1_b, mlp2_W, mlp2_b, mlp3_W, mlp3_b,
           mlpt_W, mlpt_b, fl_W1, fl_b1, fl_W2, fl_b2):
    adjacency = jnp.sum(edges, axis=3)
    mask_e = adjacency.reshape(-1) != 0
    m_mat = jnp.where(jnp.asarray(_EN_EQ) & mask_e[None, :], 1.0, 0.0)
    maskn = jnp.broadcast_to(
        (jnp.sum(adjacency, axis=-1).reshape(-1) != 0).astype(_F32)[:, None],
        (_N, _MS))
    esc = edges[0]
    mw4 = jnp.transpose(message_weights, (2, 0, 1))
    specs = [pl.BlockSpec(memory_space=pltpu.SMEM) if i == 1
             else pl.BlockSpec(memory_space=pltpu.VMEM)
             for i in range(21)]
    out = pl.pallas_call(
        _body,
        out_shape=jax.ShapeDtypeStruct((1, 989), _F32),
        in_specs=specs,
        scratch_shapes=[pltpu.VMEM((_N, _N * _MS, _MS), _BF),
                        pltpu.VMEM((_N * _N, _MS), _BF)],
    )(nodes[0], esc, mw4, m_mat, maskn,
      gru_w_ih.reshape(3, _MS, _MS), gru_w_hh.reshape(3, _MS, _MS),
      gru_b_ih.reshape(3, _MS), gru_b_hh.reshape(3, _MS),
      mlp1_W, mlp1_b, mlp2_W, mlp2_b, mlp3_W, mlp3_b, mlpt_W, mlpt_b,
      fl_W1, fl_b1.reshape(1, 500), fl_W2, fl_b2.reshape(1, 989))
    return out.reshape(989)
